# baseline (speedup 1.0000x reference)
import functools

import jax
import jax.numpy as jnp
from jax import lax
from jax.experimental import pallas as pl
from jax.experimental.pallas import tpu as pltpu
from jax.experimental.pallas import tpu_sc as plsc

B, C, T, H, W = 8, 768, 16, 14, 14
PAD = H // 7
HV = H - PAD
WV = W - 2 * PAD
NVALID = HV * WV
POS = T * C // 16

NC, NS = 2, 16
HG = 4
SLAB = 2
WSC = 2
WTC = WV - WSC
NCHUNK = (HV // HG) * (WSC // SLAB)
NBUF = 4
UNROLL = 16
TQ = T // 2

_mesh = plsc.VectorSubcoreMesh(core_axis_name="c", subcore_axis_name="s")


@functools.partial(
    pl.kernel,
    out_type=jax.ShapeDtypeStruct((B, T, C), jnp.float32),
    mesh=_mesh,
    scratch_types=[
        pltpu.VMEM((NBUF, SLAB, T, C), jnp.float32),
        pltpu.VMEM((T, C), jnp.float32),
        pltpu.VMEM_SHARED((NS, T, C), jnp.float32),
        pltpu.SemaphoreType.DMA,
        pltpu.SemaphoreType.DMA,
        pltpu.SemaphoreType.DMA,
        pltpu.SemaphoreType.DMA,
    ],
)
def _pool_kernel(x_hbm, out_hbm, buf, acc, shared, sem0, sem1, sem2, sem3):
    cid = lax.axis_index("c")
    sid = lax.axis_index("s")
    b = cid * (NS // HG) + sid // HG
    m = sid % HG
    sems = (sem0, sem1, sem2, sem3)
    inv = jnp.float32(1.0 / NVALID)

    def split(v):
        return v & (T - 1), (v >> 4) * 16

    def start_dma(k, p):
        h = 3 * m + k
        pltpu.async_copy(
            x_hbm.at[b, h, pl.ds(PAD, SLAB)], buf.at[p], sems[p]
        )

    def wait_dma(p):
        pltpu.make_async_copy(
            x_hbm.at[0, 0, pl.ds(0, SLAB)], buf.at[p], sems[p]
        ).wait()

    def accumulate(p, first):
        @plsc.parallel_loop(0, POS, unroll=UNROLL)
        def _(v):
            t, c0 = split(v)
            s = buf[p, 0, t, pl.ds(c0, 16)] + buf[p, 1, t, pl.ds(c0, 16)]
            if first:
                acc[t, pl.ds(c0, 16)] = s
            else:
                plsc.addupdate(acc.at[t, pl.ds(c0, 16)], s)

    for p in range(NBUF - 1):
        start_dma(p, p)
    for k in range(NCHUNK):
        p = k % NBUF
        wait_dma(p)
        accumulate(p, first=(k == 0))
        if k + NBUF - 1 < NCHUNK:
            start_dma(k + NBUF - 1, (k + NBUF - 1) % NBUF)

    pltpu.sync_copy(acc, shared.at[sid])
    plsc.subcore_barrier()

    @pl.when(m < 2)
    def _():
        base = (sid // HG) * HG
        t0 = m * TQ
        for j in range(HG):
            pltpu.sync_copy(
                shared.at[base + j, pl.ds(t0, TQ)],
                buf.at[j, 0, pl.ds(0, TQ)],
            )

        @plsc.parallel_loop(0, TQ * C // 16, unroll=UNROLL)
        def _(v):
            t, c0 = v & (TQ - 1), (v >> 3) * 16
            s = (
                buf[0, 0, t, pl.ds(c0, 16)] + buf[1, 0, t, pl.ds(c0, 16)]
            ) + (
                buf[2, 0, t, pl.ds(c0, 16)] + buf[3, 0, t, pl.ds(c0, 16)]
            )
            acc[t, pl.ds(c0, 16)] = s

        pltpu.sync_copy(
            acc.at[pl.ds(0, TQ)], out_hbm.at[b, pl.ds(t0, TQ)]
        )


def _tc_body(x_ref, o_ref):
    h = pl.program_id(1)
    wi = pl.program_id(2)
    s = x_ref[0, 0, 0] + x_ref[0, 0, 1]

    @pl.when((h == 0) & (wi == 0))
    def _():
        o_ref[0] = s

    @pl.when((h != 0) | (wi != 0))
    def _():
        o_ref[0] = o_ref[0] + s


_tc_pool = pl.pallas_call(
    _tc_body,
    grid=(B, HV, WTC // SLAB),
    in_specs=[
        pl.BlockSpec(
            (1, 1, SLAB, T, C),
            lambda b, h, wi: (b, h, wi + (PAD + WSC) // SLAB, 0, 0),
        )
    ],
    out_specs=pl.BlockSpec((1, T, C), lambda b, h, wi: (b, 0, 0)),
    out_shape=jax.ShapeDtypeStruct((B, T, C), jnp.float32),
    compiler_params=pltpu.CompilerParams(
        dimension_semantics=("parallel", "arbitrary", "arbitrary")
    ),
)


def kernel(x):
    xt = jnp.transpose(x, (0, 3, 4, 2, 1))
    y_sc = _pool_kernel(xt)
    y_tc = _tc_pool(xt)
    y = (y_sc + y_tc) * jnp.float32(1.0 / NVALID)
    return jnp.transpose(y, (0, 2, 1))


# speedup vs baseline: 2.7159x; 2.7159x over previous
import functools

import jax
import jax.numpy as jnp
from jax import lax
from jax.experimental import pallas as pl
from jax.experimental.pallas import tpu as pltpu
from jax.experimental.pallas import tpu_sc as plsc

B, C, T, H, W = 8, 768, 16, 14, 14
PAD = H // 7
HV = H - PAD
WV = W - 2 * PAD
NVALID = HV * WV
POS = T * C // 16

NC, NS = 2, 16
HG = 4
SLAB = 2
WSC = 2
WTC = WV - WSC
NCHUNK = (HV // HG) * (WSC // SLAB)
NBUF = 4
UNROLL = 16
TQ = T // 2

_mesh = plsc.VectorSubcoreMesh(core_axis_name="c", subcore_axis_name="s")


@functools.partial(
    pl.kernel,
    out_type=jax.ShapeDtypeStruct((B, T, C), jnp.float32),
    mesh=_mesh,
    scratch_types=[
        pltpu.VMEM((NBUF, SLAB, T, C), jnp.float32),
        pltpu.VMEM((T, C), jnp.float32),
        pltpu.VMEM_SHARED((NS, T, C), jnp.float32),
        pltpu.SemaphoreType.DMA,
        pltpu.SemaphoreType.DMA,
        pltpu.SemaphoreType.DMA,
        pltpu.SemaphoreType.DMA,
    ],
)
def _pool_kernel(x_hbm, out_hbm, buf, acc, shared, sem0, sem1, sem2, sem3):
    cid = lax.axis_index("c")
    sid = lax.axis_index("s")
    b = cid * (NS // HG) + sid // HG
    m = sid % HG
    sems = (sem0, sem1, sem2, sem3)
    inv = jnp.float32(1.0 / NVALID)

    def split(v):
        return v & (T - 1), (v >> 4) * 16

    def start_dma(k, p):
        h = 3 * m + k
        pltpu.async_copy(
            x_hbm.at[b, h, pl.ds(PAD, SLAB)], buf.at[p], sems[p]
        )

    def wait_dma(p):
        pltpu.make_async_copy(
            x_hbm.at[0, 0, pl.ds(0, SLAB)], buf.at[p], sems[p]
        ).wait()

    def accumulate(p, first):
        @plsc.parallel_loop(0, POS, unroll=UNROLL)
        def _(v):
            t, c0 = split(v)
            s = buf[p, 0, t, pl.ds(c0, 16)] + buf[p, 1, t, pl.ds(c0, 16)]
            if first:
                acc[t, pl.ds(c0, 16)] = s
            else:
                plsc.addupdate(acc.at[t, pl.ds(c0, 16)], s)

    for p in range(NBUF - 1):
        start_dma(p, p)
    for k in range(NCHUNK):
        p = k % NBUF
        wait_dma(p)
        accumulate(p, first=(k == 0))
        if k + NBUF - 1 < NCHUNK:
            start_dma(k + NBUF - 1, (k + NBUF - 1) % NBUF)

    pltpu.sync_copy(acc, shared.at[sid])
    plsc.subcore_barrier()

    @pl.when(m < 2)
    def _():
        base = (sid // HG) * HG
        t0 = m * TQ
        for j in range(HG):
            pltpu.sync_copy(
                shared.at[base + j, pl.ds(t0, TQ)],
                buf.at[j, 0, pl.ds(0, TQ)],
            )

        @plsc.parallel_loop(0, TQ * C // 16, unroll=UNROLL)
        def _(v):
            t, c0 = v & (TQ - 1), (v >> 3) * 16
            s = (
                buf[0, 0, t, pl.ds(c0, 16)] + buf[1, 0, t, pl.ds(c0, 16)]
            ) + (
                buf[2, 0, t, pl.ds(c0, 16)] + buf[3, 0, t, pl.ds(c0, 16)]
            )
            acc[t, pl.ds(c0, 16)] = s

        pltpu.sync_copy(
            acc.at[pl.ds(0, TQ)], out_hbm.at[b, pl.ds(t0, TQ)]
        )


TCH = 4


def _tc_body(x_ref, o_ref):
    hi = pl.program_id(1)
    wi = pl.program_id(2)
    s = jnp.sum(x_ref[0], axis=(0, 1))

    @pl.when((hi == 0) & (wi == 0))
    def _():
        o_ref[0] = s

    @pl.when((hi != 0) | (wi != 0))
    def _():
        o_ref[0] = o_ref[0] + s


_tc_pool = pl.pallas_call(
    _tc_body,
    grid=(B, HV // TCH, WTC // SLAB),
    in_specs=[
        pl.BlockSpec(
            (1, TCH, SLAB, T, C),
            lambda b, hi, wi: (b, hi, wi + (PAD + WSC) // SLAB, 0, 0),
        )
    ],
    out_specs=pl.BlockSpec((1, T, C), lambda b, hi, wi: (b, 0, 0)),
    out_shape=jax.ShapeDtypeStruct((B, T, C), jnp.float32),
    compiler_params=pltpu.CompilerParams(
        dimension_semantics=("arbitrary", "arbitrary", "arbitrary")
    ),
)


def kernel(x):
    xt = jnp.transpose(x, (0, 3, 4, 2, 1))
    y_sc = _pool_kernel(xt)
    y_tc = _tc_pool(xt)
    y = (y_sc + y_tc) * jnp.float32(1.0 / NVALID)
    return jnp.transpose(y, (0, 2, 1))


# speedup vs baseline: 3.9795x; 1.4652x over previous
import functools

import jax
import jax.numpy as jnp
from jax import lax
from jax.experimental import pallas as pl
from jax.experimental.pallas import tpu as pltpu
from jax.experimental.pallas import tpu_sc as plsc

B, C, T, H, W = 8, 768, 16, 14, 14
PAD = H // 7
HV = H - PAD
WV = W - 2 * PAD
NVALID = HV * WV
POS = T * C // 16

NC, NS = 2, 16
HG = 4
SLAB = 2
NCHUNK = (HV // HG) * (WV // SLAB)
NBUF = 4
UNROLL = 16
TQ = T // 2

_mesh = plsc.VectorSubcoreMesh(core_axis_name="c", subcore_axis_name="s")


@functools.partial(
    pl.kernel,
    out_type=jax.ShapeDtypeStruct((B, T, C), jnp.float32),
    mesh=_mesh,
    scratch_types=[
        pltpu.VMEM((NBUF, SLAB, T, C), jnp.float32),
        pltpu.VMEM((T, C), jnp.float32),
        pltpu.VMEM_SHARED((NS, T, C), jnp.float32),
        pltpu.SemaphoreType.DMA,
        pltpu.SemaphoreType.DMA,
        pltpu.SemaphoreType.DMA,
        pltpu.SemaphoreType.DMA,
    ],
)
def _pool_kernel(x_hbm, out_hbm, buf, acc, shared, sem0, sem1, sem2, sem3):
    cid = lax.axis_index("c")
    sid = lax.axis_index("s")
    b = cid * (NS // HG) + sid // HG
    m = sid % HG
    sems = (sem0, sem1, sem2, sem3)
    inv = jnp.float32(1.0 / NVALID)

    def split(v):
        return v & (T - 1), (v >> 4) * 16

    def start_dma(k, p):
        h = 3 * m + k // 5
        w0 = PAD + SLAB * (k % 5)
        pltpu.async_copy(x_hbm.at[b, h, pl.ds(w0, SLAB)], buf.at[p], sems[p])

    def wait_dma(p):
        pltpu.make_async_copy(
            x_hbm.at[0, 0, pl.ds(0, SLAB)], buf.at[p], sems[p]
        ).wait()

    def accumulate(p, first):
        @plsc.parallel_loop(0, POS, unroll=UNROLL)
        def _(v):
            t, c0 = split(v)
            s = buf[p, 0, t, pl.ds(c0, 16)] + buf[p, 1, t, pl.ds(c0, 16)]
            if first:
                acc[t, pl.ds(c0, 16)] = s
            else:
                plsc.addupdate(acc.at[t, pl.ds(c0, 16)], s)

    for p in range(NBUF - 1):
        start_dma(p, p)
    for k in range(NCHUNK):
        p = k % NBUF
        wait_dma(p)
        accumulate(p, first=(k == 0))
        if k + NBUF - 1 < NCHUNK:
            start_dma(k + NBUF - 1, (k + NBUF - 1) % NBUF)

    pltpu.sync_copy(acc, shared.at[sid])
    plsc.subcore_barrier()

    @pl.when(m < 2)
    def _():
        base = (sid // HG) * HG
        t0 = m * TQ
        for j in range(HG):
            pltpu.sync_copy(
                shared.at[base + j, pl.ds(t0, TQ)],
                buf.at[j, 0, pl.ds(0, TQ)],
            )

        @plsc.parallel_loop(0, TQ * C // 16, unroll=UNROLL)
        def _(v):
            t, c0 = v & (TQ - 1), (v >> 3) * 16
            s = (
                buf[0, 0, t, pl.ds(c0, 16)] + buf[1, 0, t, pl.ds(c0, 16)]
            ) + (
                buf[2, 0, t, pl.ds(c0, 16)] + buf[3, 0, t, pl.ds(c0, 16)]
            )
            acc[t, pl.ds(c0, 16)] = s * inv

        pltpu.sync_copy(
            acc.at[pl.ds(0, TQ)], out_hbm.at[b, pl.ds(t0, TQ)]
        )


def kernel(x):
    xt = jnp.transpose(x, (0, 3, 4, 2, 1))
    y = _pool_kernel(xt)
    return jnp.transpose(y, (0, 2, 1))
